# baseline (speedup 1.0000x reference)
import functools

import jax
import jax.numpy as jnp
from jax import lax
from jax.experimental import pallas as pl
from jax.experimental.pallas import tpu as pltpu
from jax.experimental.pallas import tpu_sc as plsc

_DROP_LENGTH_LOW = 1000
_DROP_LENGTH_HIGH = 8000
_DROP_COUNT_LOW = 1
_DROP_COUNT_HIGH = 10
_SEED = 42

_B = 32
_T = 160000
_MAXD = _DROP_COUNT_HIGH
_NC = 2
_NS = 16
_RS = 16
_SEG = 80000
_C = 40000
_NCH = _SEG // _C
_D = 2
_L = _D - 1
_TCB = 32000


def _interval_params(lengths):
    key = jax.random.key(_SEED)
    kp, kc, kl, ks = jax.random.split(key, 4)
    clean_length = (lengths * _T).astype(jnp.int32)
    drop_times = jax.random.randint(kc, (_B,), _DROP_COUNT_LOW, _DROP_COUNT_HIGH)
    chunk_len = jax.random.randint(
        kl, (_B, _MAXD), _DROP_LENGTH_LOW, _DROP_LENGTH_HIGH + 1)
    u = jax.random.uniform(ks, (_B, _MAXD))
    max_start = jnp.maximum(clean_length[:, None] - chunk_len, 1)
    start = (u * max_start.astype(jnp.float32)).astype(jnp.int32)
    valid = jnp.arange(_MAXD)[None, :] < drop_times[:, None]
    end = jnp.where(valid, start + chunk_len, start)
    p32 = jnp.zeros((_B, 32), jnp.int32)
    p32 = p32.at[:, :_MAXD].set(start).at[:, 16:16 + _MAXD].set(end)
    return p32


def _zero_chunk(buf, slotbase, cb, sv, ev):
    align16 = jnp.int32(-16)
    zf = jnp.zeros((16,), jnp.float32)
    for d in range(_MAXD):
        s = sv[d]
        e = ev[d]
        ls = jnp.clip(s - cb, 0, _C)
        le = jnp.clip(e - cb, 0, _C)
        ia = (ls + 15) & align16
        ib = le & align16
        n = jnp.maximum((ib - ia) >> 4, 0)

        def body(t, c, ia=ia):
            off = pl.multiple_of(slotbase + ia + t * 16, 16)
            buf[pl.ds(off, 16)] = zf
            return c

        lax.fori_loop(0, n, body, jnp.int32(0))

        for wb in (jnp.minimum(ls & align16, _C - 16),
                   jnp.minimum(ib, _C - 16)):
            wbs = pl.multiple_of(slotbase + wb, 16)
            gidx = cb + wb + lax.iota(jnp.int32, 16)
            m = (gidx >= s) & (gidx < e)
            buf[pl.ds(wbs, 16)] = jnp.where(
                m, jnp.float32(0), buf[pl.ds(wbs, 16)])


def _sc_body(w_hbm, p_hbm, out_hbm, pv_ref, buf, isems, osems):
    cid = lax.axis_index("c")
    sid = lax.axis_index("s")
    wid = sid * _NC + cid
    segbase = wid * _SEG
    half = wid % 2

    pbase = pl.multiple_of((wid // 2) * 32, 8)
    pltpu.sync_copy(p_hbm.at[pl.ds(pbase, 32)], pv_ref)
    sv = pv_ref[pl.ds(0, 16)]
    ev = pv_ref[pl.ds(16, 16)]

    def hbm_chunk(ref, j):
        return ref.at[pl.ds(pl.multiple_of(segbase + j * _C, 8), _C)]

    def slot(j):
        return buf.at[pl.ds((j % _D) * _C, _C)]

    rds = [None] * _NCH
    wrs = [None] * _NCH
    for j in range(min(_L, _NCH)):
        rds[j] = pltpu.async_copy(hbm_chunk(w_hbm, j), slot(j), isems[j % _D])
    for i in range(_NCH):
        k = i + _L
        if k < _NCH:
            if k >= _D:
                wrs[k - _D].wait()
            rds[k] = pltpu.async_copy(
                hbm_chunk(w_hbm, k), slot(k), isems[k % _D])
        rds[i].wait()
        _zero_chunk(buf, (i % _D) * _C, half * _SEG + i * _C, sv, ev)
        wrs[i] = pltpu.async_copy(slot(i), hbm_chunk(out_hbm, i),
                                  osems[i % _D])
    for i in range(max(_NCH - _D, 0), _NCH):
        wrs[i].wait()


def _tc_body(w_ref, s_ref, e_ref, o_ref):
    cols = (pl.program_id(0) * _TCB
            + lax.broadcasted_iota(jnp.int32, (_B - _RS, _TCB), 1))
    s = s_ref[...]
    e = e_ref[...]
    drop = (cols >= s[:, 0][:, None]) & (cols < e[:, 0][:, None])
    for d in range(1, _MAXD):
        drop = drop | ((cols >= s[:, d][:, None]) & (cols < e[:, d][:, None]))
    o_ref[...] = jnp.where(drop, jnp.float32(0), w_ref[...])


def kernel(waveform, lengths):
    p32 = _interval_params(lengths)
    w_sc = waveform[:_RS].reshape(-1)
    p_sc = p32[:_RS].reshape(-1)

    mesh = plsc.VectorSubcoreMesh(core_axis_name="c", subcore_axis_name="s")

    @functools.partial(
        pl.kernel,
        out_type=jax.ShapeDtypeStruct((_RS * _T,), jnp.float32),
        mesh=mesh,
        scratch_types=[
            pltpu.VMEM((32,), jnp.int32),
            pltpu.VMEM((_D * _C,), jnp.float32),
        ] + [pltpu.SemaphoreType.DMA] * (2 * _D),
    )
    def run(w_hbm, p_hbm, out_hbm, pv_ref, buf, *sems):
        _sc_body(w_hbm, p_hbm, out_hbm, pv_ref, buf, sems[:_D], sems[_D:])

    sc_out = run(w_sc, p_sc)

    tc_out = pl.pallas_call(
        _tc_body,
        grid=(_T // _TCB,),
        in_specs=[
            pl.BlockSpec((_B - _RS, _TCB), lambda j: (0, j)),
            pl.BlockSpec((_B - _RS, 16), lambda j: (0, 0)),
            pl.BlockSpec((_B - _RS, 16), lambda j: (0, 0)),
        ],
        out_specs=pl.BlockSpec((_B - _RS, _TCB), lambda j: (0, j)),
        out_shape=jax.ShapeDtypeStruct((_B - _RS, _T), jnp.float32),
    )(waveform[_RS:], p32[_RS:, :16], p32[_RS:, 16:])

    return jnp.concatenate([sc_out.reshape(_RS, _T), tc_out], axis=0)


# speedup vs baseline: 1.1265x; 1.1265x over previous
import functools

import jax
import jax.numpy as jnp
from jax import lax
from jax.experimental import pallas as pl
from jax.experimental.pallas import tpu as pltpu
from jax.experimental.pallas import tpu_sc as plsc

_DROP_LENGTH_LOW = 1000
_DROP_LENGTH_HIGH = 8000
_DROP_COUNT_LOW = 1
_DROP_COUNT_HIGH = 10
_SEED = 42

_B = 32
_T = 160000
_MAXD = _DROP_COUNT_HIGH
_NC = 2
_NS = 16
_C = 40000
_NCH = _T // _C
_D = 3
_L = _D - 1


def _interval_params(lengths):
    key = jax.random.key(_SEED)
    kp, kc, kl, ks = jax.random.split(key, 4)
    clean_length = (lengths * _T).astype(jnp.int32)
    drop_times = jax.random.randint(kc, (_B,), _DROP_COUNT_LOW, _DROP_COUNT_HIGH)
    chunk_len = jax.random.randint(
        kl, (_B, _MAXD), _DROP_LENGTH_LOW, _DROP_LENGTH_HIGH + 1)
    u = jax.random.uniform(ks, (_B, _MAXD))
    max_start = jnp.maximum(clean_length[:, None] - chunk_len, 1)
    start = (u * max_start.astype(jnp.float32)).astype(jnp.int32)
    valid = jnp.arange(_MAXD)[None, :] < drop_times[:, None]
    end = jnp.where(valid, start + chunk_len, start)
    p32 = jnp.zeros((_B, 32), jnp.int32)
    p32 = p32.at[:, :_MAXD].set(start).at[:, 16:16 + _MAXD].set(end)
    return p32.reshape(-1)


def _zero_chunk(buf, slotbase, cb, sv, ev):
    align16 = jnp.int32(-16)
    zf = jnp.zeros((16,), jnp.float32)
    for d in range(_MAXD):
        s = sv[d]
        e = ev[d]
        ls = jnp.clip(s - cb, 0, _C)
        le = jnp.clip(e - cb, 0, _C)
        ia = (ls + 15) & align16
        ib = le & align16
        n = jnp.maximum((ib - ia) >> 4, 0)

        def body(t, c, ia=ia):
            off = pl.multiple_of(slotbase + ia + t * 16, 16)
            buf[pl.ds(off, 16)] = zf
            return c

        lax.fori_loop(0, n, body, jnp.int32(0))

        for wb in (jnp.minimum(ls & align16, _C - 16),
                   jnp.minimum(ib, _C - 16)):
            wbs = pl.multiple_of(slotbase + wb, 16)
            gidx = cb + wb + lax.iota(jnp.int32, 16)
            m = (gidx >= s) & (gidx < e)
            buf[pl.ds(wbs, 16)] = jnp.where(
                m, jnp.float32(0), buf[pl.ds(wbs, 16)])


def _sc_body(w_hbm, p_hbm, out_hbm, pv_ref, buf, isems, osems):
    cid = lax.axis_index("c")
    sid = lax.axis_index("s")
    wid = sid * _NC + cid
    rowbase = wid * _T

    pbase = pl.multiple_of(wid * 32, 8)
    pltpu.sync_copy(p_hbm.at[pl.ds(pbase, 32)], pv_ref)
    sv = pv_ref[pl.ds(0, 16)]
    ev = pv_ref[pl.ds(16, 16)]

    def hbm_chunk(ref, j):
        return ref.at[pl.ds(pl.multiple_of(rowbase + j * _C, 8), _C)]

    def slot(j):
        return buf.at[pl.ds((j % _D) * _C, _C)]

    rds = [None] * _NCH
    wrs = [None] * _NCH
    for j in range(min(_L, _NCH)):
        rds[j] = pltpu.async_copy(hbm_chunk(w_hbm, j), slot(j), isems[j % _D])
    for i in range(_NCH):
        k = i + _L
        if k < _NCH:
            if k >= _D:
                wrs[k - _D].wait()
            rds[k] = pltpu.async_copy(
                hbm_chunk(w_hbm, k), slot(k), isems[k % _D])
        rds[i].wait()
        _zero_chunk(buf, (i % _D) * _C, i * _C, sv, ev)
        wrs[i] = pltpu.async_copy(slot(i), hbm_chunk(out_hbm, i),
                                  osems[i % _D])
    for i in range(max(_NCH - _D, 0), _NCH):
        wrs[i].wait()


def kernel(waveform, lengths):
    p_flat = _interval_params(lengths)
    w_flat = waveform.reshape(-1)

    mesh = plsc.VectorSubcoreMesh(core_axis_name="c", subcore_axis_name="s")

    @functools.partial(
        pl.kernel,
        out_type=jax.ShapeDtypeStruct((_B * _T,), jnp.float32),
        mesh=mesh,
        scratch_types=[
            pltpu.VMEM((32,), jnp.int32),
            pltpu.VMEM((_D * _C,), jnp.float32),
        ] + [pltpu.SemaphoreType.DMA] * (2 * _D),
    )
    def run(w_hbm, p_hbm, out_hbm, pv_ref, buf, *sems):
        _sc_body(w_hbm, p_hbm, out_hbm, pv_ref, buf, sems[:_D], sems[_D:])

    out = run(w_flat, p_flat)
    return out.reshape(_B, _T)
